# baseline (speedup 1.0000x reference)
import functools

import jax
import jax.numpy as jnp
from jax import lax
from jax.experimental import pallas as pl
from jax.experimental.pallas import tpu as pltpu
from jax.experimental.pallas import tpu_sc as plsc

D = 64
NC = 2
NS = 16
NW = NC * NS
SUB = 128
NSUB = 2
C = SUB * NSUB
L16 = 16
TP = 257


def _transpose_chunk(rows, tps, posrow):
    lanes = lax.iota(jnp.int32, L16)
    pj = [posrow[pl.ds(j * L16, L16)] for j in range(D // L16)]
    dts = [(lanes + j * L16) // 8 for j in range(D // L16)]
    sbs = [(lanes + j * L16) % 8 for j in range(D // L16)]

    @plsc.parallel_loop(0, C, unroll=8)
    def body(bb):
        col = lax.broadcast(bb, (L16,))
        for j in range(D // L16):
            v = rows[bb, pl.ds(j * L16, L16)] + pj[j]
            plsc.store_scatter(tps, [dts[j], sbs[j], col], v)


BLK = 256
RP = 65


def _relayout_block(src, dst):
    lanes = lax.iota(jnp.int32, L16)

    @plsc.parallel_loop(0, D, unroll=8)
    def body(dd):
        dv = lax.broadcast(dd, (L16,))
        for cb in range(BLK // L16):
            v = src[dd, pl.ds(cb * L16, L16)]
            plsc.store_scatter(dst, [lanes + cb * L16, dv], v)


def _relayout_body(wt_hbm, wtab_hbm, s0, s1, r0, r1,
                   sem_s0, sem_s1, sem_w0, sem_w1):
    V = wt_hbm.shape[1]
    nblk_pad = 2 * ((V + 2 * BLK - 1) // (2 * BLK))
    pb = (nblk_pad + NW - 1) // NW
    pb = pb + (pb % 2)
    wid = lax.axis_index("c") * NS + lax.axis_index("s")
    nv = jnp.minimum(pb, jnp.maximum(nblk_pad - wid * pb, 0))
    S = (s0, s1)
    R = (r0, r1)
    sem_s = (sem_s0, sem_s1)
    sem_w = (sem_w0, sem_w1)

    def c0_of(i):
        return jnp.minimum((wid * pb + i) * BLK, V - BLK)

    def fire(i, b):
        pltpu.async_copy(wt_hbm.at[:, pl.ds(c0_of(i), BLK)], S[b], sem_s[b])

    def drain_src(b):
        pltpu.make_async_copy(wt_hbm.at[:, pl.ds(0, BLK)], S[b],
                              sem_s[b]).wait()

    def finish(i, b):
        _relayout_block(S[b], R[b])
        pltpu.async_copy(R[b].at[:, pl.ds(0, D)],
                         wtab_hbm.at[pl.ds(c0_of(i), BLK)], sem_w[b])

    def wait_wb(b):
        pltpu.make_async_copy(R[b].at[:, pl.ds(0, D)],
                              wtab_hbm.at[pl.ds(0, BLK)], sem_w[b]).wait()

    @pl.when(nv > 0)
    def _():
        fire(0, 0)

    def pair(t, carry):
        i0 = 2 * t

        @pl.when(i0 + 1 < nv)
        def _():
            fire(i0 + 1, 1)

        @pl.when(i0 < nv)
        def _():
            drain_src(0)

        @pl.when((t >= 1) & (i0 < nv))
        def _():
            wait_wb(0)

        @pl.when(i0 < nv)
        def _():
            finish(i0, 0)

        @pl.when(i0 + 2 < nv)
        def _():
            fire(i0 + 2, 0)

        @pl.when(i0 + 1 < nv)
        def _():
            drain_src(1)

        @pl.when((t >= 1) & (i0 + 1 < nv))
        def _():
            wait_wb(1)

        @pl.when(i0 + 1 < nv)
        def _():
            finish(i0 + 1, 1)

        return carry

    lax.fori_loop(0, pb // 2, pair, 0)

    @pl.when(nv >= 2)
    def _():
        wait_wb(0)
        wait_wb(1)


@jax.jit
def _relayout(word_t):
    mesh = plsc.VectorSubcoreMesh(core_axis_name="c", subcore_axis_name="s",
                                  num_cores=NC, num_subcores=NS)
    return pl.kernel(
        _relayout_body,
        out_type=jax.ShapeDtypeStruct((word_t.shape[1], D), jnp.float32),
        mesh=mesh,
        compiler_params=pltpu.CompilerParams(use_tc_tiling_on_sc=False,
                                             needs_layout_passes=False),
        scratch_types=[
            pltpu.VMEM((D, BLK), jnp.float32),
            pltpu.VMEM((D, BLK), jnp.float32),
            pltpu.VMEM((BLK, RP), jnp.float32),
            pltpu.VMEM((BLK, RP), jnp.float32),
            pltpu.SemaphoreType.DMA,
            pltpu.SemaphoreType.DMA,
            pltpu.SemaphoreType.DMA,
            pltpu.SemaphoreType.DMA,
        ],
    )(word_t)


def _emb_body(word_hbm, pos_hbm, xt_hbm, out_hbm,
              idx0, idx1, rows0, rows1, tps0, tps1, pos0, pos1,
              sem_g0, sem_g1, sem_w0, sem_w1):
    n_l = out_hbm.shape[0]
    n_b = out_hbm.shape[2] * 128
    n_rows = n_l * n_b
    per_w = n_rows // NW
    chunks = per_w // C
    idx_rows_per_w = per_w // SUB
    wid = lax.axis_index("c") * NS + lax.axis_index("s")
    base = wid * per_w
    idx = (idx0, idx1)
    rows = (rows0, rows1)
    tps = (tps0, tps1)
    pos = (pos0, pos1)
    sem_g = (sem_g0, sem_g1)
    sem_w = (sem_w0, sem_w1)

    def fire(k, b):
        pltpu.sync_copy(xt_hbm.at[pl.ds(wid * idx_rows_per_w + k * NSUB,
                                        NSUB)], idx[b])
        l = (base + k * C) // n_b
        pltpu.sync_copy(pos_hbm.at[l], pos[b])
        for j in range(NSUB):
            pltpu.async_copy(word_hbm.at[idx[b].at[j]],
                             rows[b].at[pl.ds(j * SUB, SUB)], sem_g[b])

    def drain_gather(b):
        pltpu.make_async_copy(word_hbm.at[pl.ds(0, C)], rows[b],
                              sem_g[b]).wait()

    def finish(k, b):
        _transpose_chunk(rows[b], tps[b], pos[b])
        flat = base + k * C
        l = flat // n_b
        bt0 = (flat - l * n_b) // 128
        for t in range(C // 128):
            pltpu.async_copy(tps[b].at[:, :, pl.ds(t * 128, 128)],
                             out_hbm.at[l, :, bt0 + t], sem_w[b])

    def wait_wb(b):
        pltpu.make_async_copy(tps[b].at[:, :, pl.ds(0, C)],
                              out_hbm.at[0, :, pl.ds(0, C // 128)],
                              sem_w[b]).wait()

    fire(0, 0)

    def pair(t, carry):
        k0 = 2 * t
        fire(k0 + 1, 1)
        drain_gather(0)

        @pl.when(t >= 1)
        def _():
            wait_wb(0)

        finish(k0, 0)

        @pl.when(t < chunks // 2 - 1)
        def _():
            fire(k0 + 2, 0)

        drain_gather(1)

        @pl.when(t >= 1)
        def _():
            wait_wb(1)

        finish(k0 + 1, 1)
        return carry

    lax.fori_loop(0, chunks // 2, pair, 0)
    wait_wb(0)
    wait_wb(1)


@functools.partial(jax.jit, static_argnames=("n_l", "n_b"))
def _emb(word_table, pos_table, xt2d, n_l, n_b):
    mesh = plsc.VectorSubcoreMesh(core_axis_name="c", subcore_axis_name="s",
                                  num_cores=NC, num_subcores=NS)
    return pl.kernel(
        _emb_body,
        out_type=jax.ShapeDtypeStruct((n_l, D // 8, n_b // 128, 8, 128),
                                      jnp.float32),
        mesh=mesh,
        compiler_params=pltpu.CompilerParams(use_tc_tiling_on_sc=False,
                                             needs_layout_passes=False),
        scratch_types=[
            pltpu.VMEM((NSUB, SUB), jnp.int32),
            pltpu.VMEM((NSUB, SUB), jnp.int32),
            pltpu.VMEM((C, D), jnp.float32),
            pltpu.VMEM((C, D), jnp.float32),
            pltpu.VMEM((D // 8, 8, TP), jnp.float32),
            pltpu.VMEM((D // 8, 8, TP), jnp.float32),
            pltpu.VMEM((D,), jnp.float32),
            pltpu.VMEM((D,), jnp.float32),
            pltpu.SemaphoreType.DMA,
            pltpu.SemaphoreType.DMA,
            pltpu.SemaphoreType.DMA,
            pltpu.SemaphoreType.DMA,
        ],
    )(word_table, pos_table, xt2d)


def kernel(word_table, pos_table, x):
    Bx, Lx = x.shape
    n_rows = Bx * Lx
    xt2d = x.T.reshape(n_rows // SUB, SUB).astype(jnp.int32)
    wtab = _relayout(word_table.T)
    out5 = _emb(wtab, pos_table, xt2d, Lx, Bx)
    return out5.transpose(2, 4, 0, 1, 3).reshape(Bx, Lx, D)


# speedup vs baseline: 7.1392x; 7.1392x over previous
import functools

import jax
import jax.numpy as jnp
from jax import lax
from jax.experimental import pallas as pl
from jax.experimental.pallas import tpu as pltpu
from jax.experimental.pallas import tpu_sc as plsc

D = 64
NC = 2
NS = 16
NW = NC * NS
SUB = 128
NSUB = 2
C = SUB * NSUB
L16 = 16
TP = 257
NR = 4


def _transpose_chunk(rows, tps, pos_all, l):
    lanes = lax.iota(jnp.int32, L16)
    pj = [pos_all[l, pl.ds(j * L16, L16)] for j in range(D // L16)]
    dts = [(lanes + j * L16) // 8 for j in range(D // L16)]
    sbs = [(lanes + j * L16) % 8 for j in range(D // L16)]

    @plsc.parallel_loop(0, C, unroll=8)
    def body(bb):
        col = lax.broadcast(bb, (L16,))
        for j in range(D // L16):
            v = rows[bb, pl.ds(j * L16, L16)] + pj[j]
            plsc.store_scatter(tps, [dts[j], sbs[j], col], v)


def _emb_body(word_hbm, pos_hbm, xt_hbm, out_hbm,
              idx0, idx1, idx2, idx3, rows0, rows1, rows2, rows3,
              tps0, tps1, pos_all,
              sem_g0, sem_g1, sem_g2, sem_g3,
              sem_i0, sem_i1, sem_i2, sem_i3, sem_w0, sem_w1):
    n_l = out_hbm.shape[0]
    n_b = out_hbm.shape[2] * 128
    n_rows = n_l * n_b
    per_w = n_rows // NW
    chunks = per_w // C
    idx_rows_per_w = per_w // SUB
    wid = lax.axis_index("c") * NS + lax.axis_index("s")
    base = wid * per_w
    idx = (idx0, idx1, idx2, idx3)
    rows = (rows0, rows1, rows2, rows3)
    tps = (tps0, tps1)
    sem_g = (sem_g0, sem_g1, sem_g2, sem_g3)
    sem_i = (sem_i0, sem_i1, sem_i2, sem_i3)
    sem_w = (sem_w0, sem_w1)

    pltpu.sync_copy(pos_hbm, pos_all)

    def fire_idx(k, b):
        pltpu.async_copy(xt_hbm.at[pl.ds(wid * idx_rows_per_w + k * NSUB,
                                         NSUB)], idx[b], sem_i[b])

    def fire(k, b):
        pltpu.make_async_copy(xt_hbm.at[pl.ds(0, NSUB)], idx[b],
                              sem_i[b]).wait()
        for j in range(NSUB):
            pltpu.async_copy(word_hbm.at[idx[b].at[j]],
                             rows[b].at[pl.ds(j * SUB, SUB)], sem_g[b])

    def drain_gather(b):
        pltpu.make_async_copy(word_hbm.at[pl.ds(0, C)], rows[b],
                              sem_g[b]).wait()

    def finish(k, rb, tb):
        flat = base + k * C
        l = flat // n_b
        _transpose_chunk(rows[rb], tps[tb], pos_all, l)
        bt0 = (flat - l * n_b) // 128
        for t in range(C // 128):
            pltpu.async_copy(tps[tb].at[:, :, pl.ds(t * 128, 128)],
                             out_hbm.at[l, :, bt0 + t], sem_w[tb])

    def wait_wb(b):
        pltpu.make_async_copy(tps[b].at[:, :, pl.ds(0, C)],
                              out_hbm.at[0, :, pl.ds(0, C // 128)],
                              sem_w[b]).wait()

    fire_idx(0, 0)
    fire_idx(1, 1)
    fire(0, 0)
    fire(1, 1)
    fire_idx(2, 2)
    fire_idx(3, 3)

    def quad(t, carry):
        for u in range(NR):
            k = NR * t + u
            kf = k + 2

            @pl.when(kf < chunks)
            def _():
                fire(kf, (u + 2) % NR)

            drain_gather(u)

            @pl.when(k + NR < chunks)
            def _():
                fire_idx(k + NR, u)

            @pl.when(k >= 2)
            def _():
                wait_wb(u % 2)

            finish(k, u, u % 2)
        return carry

    lax.fori_loop(0, chunks // NR, quad, 0)
    wait_wb(0)
    wait_wb(1)


@functools.partial(jax.jit, static_argnames=("n_l", "n_b"))
def _emb(word_table, pos_table, xt2d, n_l, n_b):
    mesh = plsc.VectorSubcoreMesh(core_axis_name="c", subcore_axis_name="s",
                                  num_cores=NC, num_subcores=NS)
    return pl.kernel(
        _emb_body,
        out_type=jax.ShapeDtypeStruct((n_l, D // 8, n_b // 128, 8, 128),
                                      jnp.float32),
        mesh=mesh,
        compiler_params=pltpu.CompilerParams(use_tc_tiling_on_sc=False,
                                             needs_layout_passes=False),
        scratch_types=(
            [pltpu.VMEM((NSUB, SUB), jnp.int32) for _ in range(NR)]
            + [pltpu.VMEM((C, D), jnp.float32) for _ in range(NR)]
            + [pltpu.VMEM((D // 8, 8, TP), jnp.float32) for _ in range(2)]
            + [pltpu.VMEM((200, D), jnp.float32)]
            + [pltpu.SemaphoreType.DMA for _ in range(NR)]
            + [pltpu.SemaphoreType.DMA for _ in range(NR)]
            + [pltpu.SemaphoreType.DMA for _ in range(2)]
        ),
    )(word_table, pos_table, xt2d)


def kernel(word_table, pos_table, x):
    Bx, Lx = x.shape
    n_rows = Bx * Lx
    xt2d = x.T.reshape(n_rows // SUB, SUB).astype(jnp.int32)
    out5 = _emb(word_table, pos_table, xt2d, Lx, Bx)
    return out5.transpose(2, 4, 0, 1, 3).reshape(Bx, Lx, D)
